# Initial kernel scaffold; baseline (speedup 1.0000x reference)
import jax
import jax.numpy as jnp
from jax.experimental import pallas as pl


def kernel(x, edge_index, batch, W_conv, b_conv, W1, b1, W2, b2, W_out, b_out):
    raise NotImplementedError("write your pallas kernel here")


# speedup vs baseline: 39.8119x; 39.8119x over previous
import functools

import jax
import jax.numpy as jnp
from jax import lax
from jax.experimental import pallas as pl
from jax.experimental.pallas import tpu as pltpu
from jax.experimental.pallas import tpu_sc as plsc

N_NODES = 10000
N_EDGES = 320000
N_GRAPHS = 100
D_FEAT = 128
HID = 16
N_MIC = 10
N_ACTS = 8

NC = 2
NS = 16
LANES = 16
NW = NC * NS
CH = 128
ROWS_PER_W = 80
N_ROWS = NW * ROWS_PER_W
E_PAD = N_ROWS * CH
NPAD = 10016
DEGPAD = 10240
G_PAD = 128

_HIGH = lax.Precision.HIGHEST


def _sc_deg_body(dst_rows, zeros_deg, deg_out, idx_v, ones_v, sh_deg):
    c = lax.axis_index("c")
    s = lax.axis_index("s")

    @pl.when(s == 0)
    def _init():
        pltpu.sync_copy(zeros_deg, sh_deg)

    for i in range(CH // LANES):
        ones_v[pl.ds(i * LANES, LANES)] = jnp.full((LANES,), 1.0, jnp.float32)

    wid = s * NC + c
    pltpu.sync_copy(dst_rows.at[pl.ds(wid * ROWS_PER_W, ROWS_PER_W)], idx_v)
    plsc.subcore_barrier()

    def step(j, carry):
        pltpu.sync_copy(ones_v, sh_deg.at[idx_v.at[j]], add=True)
        return carry

    lax.fori_loop(0, ROWS_PER_W, step, 0)
    plsc.subcore_barrier()

    @pl.when(s == 0)
    def _flush():
        pltpu.sync_copy(sh_deg, deg_out.at[c])


def _sc_agg_body(src_rows, dst_rows, xn, zeros_agg, agg_out,
                 sidx, didx, rows_v, sh_acc, sem):
    c = lax.axis_index("c")
    s = lax.axis_index("s")

    @pl.when(s == 0)
    def _init():
        pltpu.sync_copy(zeros_agg, sh_acc)

    wid = s * NC + c
    pltpu.sync_copy(src_rows.at[pl.ds(wid * ROWS_PER_W, ROWS_PER_W)], sidx)
    pltpu.sync_copy(dst_rows.at[pl.ds(wid * ROWS_PER_W, ROWS_PER_W)], didx)
    plsc.subcore_barrier()

    def step(j, carry):
        pltpu.async_copy(xn.at[sidx.at[j]], rows_v, sem).wait()
        pltpu.sync_copy(rows_v, sh_acc.at[didx.at[j]], add=True)
        return carry

    lax.fori_loop(0, ROWS_PER_W, step, 0)
    plsc.subcore_barrier()

    @pl.when(s == 0)
    def _flush():
        pltpu.sync_copy(sh_acc, agg_out.at[c])


def _tc_xn_body(x_ref, w_ref, degc_ref, xn_ref):
    deg = degc_ref[0, :N_NODES, :] + degc_ref[1, :N_NODES, :] + 1.0
    dinv = lax.rsqrt(deg)
    xw = jnp.dot(x_ref[...], w_ref[...],
                 preferred_element_type=jnp.float32, precision=_HIGH)
    xn_ref[...] = xw * dinv


def _elu(v):
    return jnp.where(v > 0.0, v, jnp.exp(jnp.minimum(v, 0.0)) - 1.0)


def _tc_head_body(aggp_ref, xn_ref, degc_ref, bconv_ref, batch_ref,
                  w1_ref, b1_ref, w2_ref, b2_ref, wout_ref, bout_ref,
                  out_ref):
    deg = degc_ref[0, :N_NODES, :] + degc_ref[1, :N_NODES, :] + 1.0
    dinv = lax.rsqrt(deg)
    xn = xn_ref[...]
    agg = aggp_ref[0, :N_NODES, :] + aggp_ref[1, :N_NODES, :]
    h = jnp.maximum(dinv * (agg + xn) + bconv_ref[...], 0.0)

    gid = lax.broadcasted_iota(jnp.int32, (G_PAD, N_NODES), 0)
    ohT = (gid == batch_ref[...]).astype(jnp.float32)
    sums = lax.dot_general(ohT, h, (((1,), (0,)), ((), ())),
                           preferred_element_type=jnp.float32,
                           precision=_HIGH)
    cnt = jnp.sum(ohT, axis=1, keepdims=True)
    g = sums / jnp.maximum(cnt, 1.0)

    g = _elu(jnp.dot(g, w1_ref[...],
                     preferred_element_type=jnp.float32, precision=_HIGH)
             + b1_ref[...])
    g = _elu(jnp.dot(g, w2_ref[...],
                     preferred_element_type=jnp.float32, precision=_HIGH)
             + b2_ref[...])
    out_ref[...] = jnp.dot(g, wout_ref[...],
                           preferred_element_type=jnp.float32,
                           precision=_HIGH) + bout_ref[...]


def kernel(x, edge_index, batch, W_conv, b_conv, W1, b1, W2, b2, W_out, b_out):
    ei = edge_index.astype(jnp.int32)
    src = jnp.concatenate(
        [ei[0], jnp.zeros((E_PAD - N_EDGES,), jnp.int32)]).reshape(N_ROWS, CH)
    dst = jnp.concatenate(
        [ei[1], jnp.full((E_PAD - N_EDGES,), N_NODES, jnp.int32)]
    ).reshape(N_ROWS, CH)
    zeros_deg = jnp.zeros((DEGPAD,), jnp.float32)
    zeros_agg = jnp.zeros((NPAD, HID), jnp.float32)

    mesh = plsc.VectorSubcoreMesh(core_axis_name="c", subcore_axis_name="s",
                                  num_cores=NC, num_subcores=NS)
    sc_params = pltpu.CompilerParams(use_tc_tiling_on_sc=False)

    deg_parts = pl.kernel(
        _sc_deg_body,
        out_type=jax.ShapeDtypeStruct((NC, DEGPAD), jnp.float32),
        mesh=mesh,
        scratch_types=[
            pltpu.VMEM((ROWS_PER_W, CH), jnp.int32),
            pltpu.VMEM((CH,), jnp.float32),
            pltpu.VMEM_SHARED((DEGPAD,), jnp.float32),
        ],
        compiler_params=sc_params,
    )(dst, zeros_deg)
    deg_col = deg_parts.reshape(NC, DEGPAD, 1)

    xn = pl.pallas_call(
        _tc_xn_body,
        out_shape=jax.ShapeDtypeStruct((N_NODES, HID), jnp.float32),
    )(x, W_conv, deg_col)

    agg_parts = pl.kernel(
        _sc_agg_body,
        out_type=jax.ShapeDtypeStruct((NC, NPAD, HID), jnp.float32),
        mesh=mesh,
        scratch_types=[
            pltpu.VMEM((ROWS_PER_W, CH), jnp.int32),
            pltpu.VMEM((ROWS_PER_W, CH), jnp.int32),
            pltpu.VMEM((CH, HID), jnp.float32),
            pltpu.VMEM_SHARED((NPAD, HID), jnp.float32),
            pltpu.SemaphoreType.DMA,
        ],
        compiler_params=sc_params,
    )(src, dst, xn, zeros_agg)

    batch2 = batch.astype(jnp.int32).reshape(1, N_NODES)
    woutr = W_out.transpose(1, 0, 2).reshape(HID * 4, N_MIC * N_ACTS)
    boutr = b_out.reshape(1, N_MIC * N_ACTS)

    outp = pl.pallas_call(
        _tc_head_body,
        out_shape=jax.ShapeDtypeStruct((G_PAD, N_MIC * N_ACTS), jnp.float32),
    )(agg_parts, xn, deg_col, b_conv.reshape(1, HID), batch2,
      W1, b1.reshape(1, 64), W2, b2.reshape(1, 64), woutr, boutr)

    return outp[:N_GRAPHS].reshape(N_GRAPHS, N_MIC, N_ACTS)


# speedup vs baseline: 49.2254x; 1.2364x over previous
import jax
import jax.numpy as jnp
from jax import lax
from jax.experimental import pallas as pl
from jax.experimental.pallas import tpu as pltpu
from jax.experimental.pallas import tpu_sc as plsc

N_NODES = 10000
N_EDGES = 320000
N_GRAPHS = 100
D_FEAT = 128
HID = 16
N_MIC = 10
N_ACTS = 8

NC = 2
NS = 16
LANES = 16
NW = NC * NS
CH = 128
ROWS_PER_W = 80
N_ROWS = NW * ROWS_PER_W
E_PAD = N_ROWS * CH
NPAD = 10016
G_PAD = 128

_HIGH = lax.Precision.HIGHEST


def _sc_deg_body(eip, zeros16, ones16, deg_out, idx_v, ones_v, sh_deg):
    c = lax.axis_index("c")
    s = lax.axis_index("s")

    @pl.when(s == 0)
    def _init():
        pltpu.sync_copy(zeros16, sh_deg)

    pltpu.sync_copy(ones16, ones_v)
    wid = s * NC + c
    pltpu.sync_copy(eip.at[1].at[pl.ds(wid * ROWS_PER_W, ROWS_PER_W)], idx_v)
    plsc.subcore_barrier()

    def step(j, carry):
        pltpu.sync_copy(ones_v, sh_deg.at[idx_v.at[j]], add=True)
        return carry

    lax.fori_loop(0, ROWS_PER_W, step, 0)
    plsc.subcore_barrier()

    @pl.when(s == 0)
    def _flush():
        pltpu.sync_copy(sh_deg, deg_out.at[c])


def _sc_agg_body(eip, xn, zeros16, agg_out,
                 sidx, didx, buf_a, buf_b, sh_acc, sem_a, sem_b):
    c = lax.axis_index("c")
    s = lax.axis_index("s")

    @pl.when(s == 0)
    def _init():
        pltpu.sync_copy(zeros16, sh_acc)

    wid = s * NC + c
    pltpu.sync_copy(eip.at[0].at[pl.ds(wid * ROWS_PER_W, ROWS_PER_W)], sidx)
    pltpu.sync_copy(eip.at[1].at[pl.ds(wid * ROWS_PER_W, ROWS_PER_W)], didx)
    plsc.subcore_barrier()

    pltpu.async_copy(xn.at[sidx.at[0]], buf_a, sem_a)

    def step(t, carry):
        j = 2 * t
        pltpu.async_copy(xn.at[sidx.at[j + 1]], buf_b, sem_b)
        pltpu.make_async_copy(xn.at[sidx.at[j]], buf_a, sem_a).wait()
        pltpu.sync_copy(buf_a, sh_acc.at[didx.at[j]], add=True)

        @pl.when(t + 1 < ROWS_PER_W // 2)
        def _next():
            pltpu.async_copy(xn.at[sidx.at[j + 2]], buf_a, sem_a)

        pltpu.make_async_copy(xn.at[sidx.at[j + 1]], buf_b, sem_b).wait()
        pltpu.sync_copy(buf_b, sh_acc.at[didx.at[j + 1]], add=True)
        return carry

    lax.fori_loop(0, ROWS_PER_W // 2, step, 0)
    plsc.subcore_barrier()

    @pl.when(s == 0)
    def _flush():
        pltpu.sync_copy(sh_acc, agg_out.at[c])


def _tc_xn_body(x_ref, w_ref, degp_ref, xn_ref):
    deg = degp_ref[0, :N_NODES, :] + degp_ref[1, :N_NODES, :] + 1.0
    dinv = lax.rsqrt(deg)
    xw = jnp.dot(x_ref[...], w_ref[...],
                 preferred_element_type=jnp.float32, precision=_HIGH)
    xn_ref[:N_NODES, :] = xw * dinv
    xn_ref[N_NODES:, :] = jnp.zeros((NPAD - N_NODES, HID), jnp.float32)


def _elu(v):
    return jnp.where(v > 0.0, v, jnp.exp(jnp.minimum(v, 0.0)) - 1.0)


def _tc_head_body(aggp_ref, xn_ref, degp_ref, bconv_ref, batch_ref,
                  w1_ref, b1_ref, w2_ref, b2_ref, wout_ref, bout_ref,
                  out_ref):
    deg = degp_ref[0, :N_NODES, :] + degp_ref[1, :N_NODES, :] + 1.0
    dinv = lax.rsqrt(deg)
    xn = xn_ref[:N_NODES, :]
    agg = aggp_ref[0, :N_NODES, :] + aggp_ref[1, :N_NODES, :]
    h = jnp.maximum(dinv * (agg + xn) + bconv_ref[...], 0.0)

    gid = lax.broadcasted_iota(jnp.int32, (G_PAD, N_NODES), 0)
    ohT = (gid == batch_ref[...]).astype(jnp.float32)
    sums = lax.dot_general(ohT, h, (((1,), (0,)), ((), ())),
                           preferred_element_type=jnp.float32,
                           precision=_HIGH)
    cnt = jnp.sum(ohT, axis=1, keepdims=True)
    g = sums / jnp.maximum(cnt, 1.0)

    g = _elu(jnp.dot(g, w1_ref[...],
                     preferred_element_type=jnp.float32, precision=_HIGH)
             + b1_ref[...])
    g = _elu(jnp.dot(g, w2_ref[...],
                     preferred_element_type=jnp.float32, precision=_HIGH)
             + b2_ref[...])
    out_ref[...] = jnp.dot(g, wout_ref[...],
                           preferred_element_type=jnp.float32,
                           precision=_HIGH) + bout_ref[...]


def kernel(x, edge_index, batch, W_conv, b_conv, W1, b1, W2, b2, W_out, b_out):
    eip = jnp.pad(edge_index.astype(jnp.int32),
                  ((0, 0), (0, E_PAD - N_EDGES)),
                  constant_values=N_NODES).reshape(2, N_ROWS, CH)
    zeros16 = jnp.zeros((NPAD, HID), jnp.float32)
    ones16 = jnp.ones((CH, HID), jnp.float32)

    mesh = plsc.VectorSubcoreMesh(core_axis_name="c", subcore_axis_name="s",
                                  num_cores=NC, num_subcores=NS)
    sc_params = pltpu.CompilerParams(use_tc_tiling_on_sc=False)

    deg_parts = pl.kernel(
        _sc_deg_body,
        out_type=jax.ShapeDtypeStruct((NC, NPAD, HID), jnp.float32),
        mesh=mesh,
        scratch_types=[
            pltpu.VMEM((ROWS_PER_W, CH), jnp.int32),
            pltpu.VMEM((CH, HID), jnp.float32),
            pltpu.VMEM_SHARED((NPAD, HID), jnp.float32),
        ],
        compiler_params=sc_params,
    )(eip, zeros16, ones16)

    xn = pl.pallas_call(
        _tc_xn_body,
        out_shape=jax.ShapeDtypeStruct((NPAD, HID), jnp.float32),
    )(x, W_conv, deg_parts)

    agg_parts = pl.kernel(
        _sc_agg_body,
        out_type=jax.ShapeDtypeStruct((NC, NPAD, HID), jnp.float32),
        mesh=mesh,
        scratch_types=[
            pltpu.VMEM((ROWS_PER_W, CH), jnp.int32),
            pltpu.VMEM((ROWS_PER_W, CH), jnp.int32),
            pltpu.VMEM((CH, HID), jnp.float32),
            pltpu.VMEM((CH, HID), jnp.float32),
            pltpu.VMEM_SHARED((NPAD, HID), jnp.float32),
            pltpu.SemaphoreType.DMA,
            pltpu.SemaphoreType.DMA,
        ],
        compiler_params=sc_params,
    )(eip, xn, zeros16)

    batch2 = batch.astype(jnp.int32).reshape(1, N_NODES)
    woutr = W_out.transpose(1, 0, 2).reshape(HID * 4, N_MIC * N_ACTS)
    boutr = b_out.reshape(1, N_MIC * N_ACTS)

    outp = pl.pallas_call(
        _tc_head_body,
        out_shape=jax.ShapeDtypeStruct((G_PAD, N_MIC * N_ACTS), jnp.float32),
    )(agg_parts, xn, deg_parts, b_conv.reshape(1, HID), batch2,
      W1, b1.reshape(1, 64), W2, b2.reshape(1, 64), woutr, boutr)

    return outp[:N_GRAPHS].reshape(N_GRAPHS, N_MIC, N_ACTS)
